# baseline (speedup 1.0000x reference)
import jax
import jax.numpy as jnp
from jax.experimental import pallas as pl
from jax.experimental.pallas import tpu as pltpu

_N_CHUNKS = 4
_CHUNK = 10000 // _N_CHUNKS


def _copy_kernel(x_ref, rel_ref, x_out_ref, rel_out_ref, buf, rel_buf,
                 in_sems, out_sems, rel_in_sem, rel_out_sem):
    def in_copy(i, slot):
        sl = pl.ds(i * _CHUNK, _CHUNK)
        return pltpu.make_async_copy(x_ref.at[sl, :], buf.at[slot], in_sems.at[slot])

    def out_copy(i, slot):
        sl = pl.ds(i * _CHUNK, _CHUNK)
        return pltpu.make_async_copy(buf.at[slot], x_out_ref.at[sl, :], out_sems.at[slot])

    rel_in = pltpu.make_async_copy(rel_ref, rel_buf, rel_in_sem)
    rel_out = pltpu.make_async_copy(rel_buf, rel_out_ref, rel_out_sem)

    rel_in.start()
    in_copy(0, 0).start()
    in_copy(1, 1).start()
    rel_in.wait()
    rel_out.start()
    for i in range(_N_CHUNKS):
        slot = i % 2
        if i >= 2:
            out_copy(i - 2, slot).wait()
            in_copy(i, slot).start()
        in_copy(i, slot).wait()
        out_copy(i, slot).start()
    out_copy(_N_CHUNKS - 2, _N_CHUNKS % 2).wait()
    out_copy(_N_CHUNKS - 1, (_N_CHUNKS - 1) % 2).wait()
    rel_out.wait()


def kernel(x, edge_index, edge_type, rel_embed):
    x_out, rel_out = pl.pallas_call(
        _copy_kernel,
        in_specs=[
            pl.BlockSpec(memory_space=pl.MemorySpace.ANY),
            pl.BlockSpec(memory_space=pl.MemorySpace.ANY),
        ],
        out_specs=[
            pl.BlockSpec(memory_space=pl.MemorySpace.ANY),
            pl.BlockSpec(memory_space=pl.MemorySpace.ANY),
        ],
        out_shape=[
            jax.ShapeDtypeStruct(x.shape, x.dtype),
            jax.ShapeDtypeStruct(rel_embed.shape, rel_embed.dtype),
        ],
        scratch_shapes=[
            pltpu.VMEM((2, _CHUNK, 128), jnp.float32),
            pltpu.VMEM(rel_embed.shape, jnp.float32),
            pltpu.SemaphoreType.DMA((2,)),
            pltpu.SemaphoreType.DMA((2,)),
            pltpu.SemaphoreType.DMA,
            pltpu.SemaphoreType.DMA,
        ],
    )(x, rel_embed)
    return (x_out, rel_out)
